# Initial kernel scaffold; baseline (speedup 1.0000x reference)
import jax
import jax.numpy as jnp
from jax.experimental import pallas as pl


def kernel(Px, W, gene_sf, scale_factor_visium, gamma, W_gat, a_src, a_dst, b_gat, edge_index):
    raise NotImplementedError("write your pallas kernel here")


# speedup vs baseline: 41.1358x; 41.1358x over previous
import functools

import jax
import jax.numpy as jnp
from jax import lax
from jax.experimental import pallas as pl
from jax.experimental.pallas import tpu as pltpu
from jax.experimental.pallas import tpu_sc as plsc

NXE = 10000
NVI = 2000
NF = 64
NG = 512
NE = 400000

NCORE = 2
NSUB = 16
NW = NCORE * NSUB
CH = 128
NCH = 98
EPT = NCH * CH
EPAD = NW * EPT
RPAD = 10112
STRIPE = RPAD // NSUB
REM = STRIPE - 4 * CH


DW = 16


def _sc_gat(src_h, dst_h, as_h, q_h, h_h, g_h, num_o, den_o,
            idxs_v, idxd_v, asg_a, qg_a, ex_a, exw_v, rows_g, rows_s, gv_v,
            num_sh, den_sh, sem_g0, sem_g1, sem_s0, sem_s1, sem_i):
    c = lax.axis_index("c")
    s = lax.axis_index("s")
    wid = c * NSUB + s
    r0 = s * STRIPE

    pltpu.sync_copy(src_h.at[wid], idxs_v)
    pltpu.sync_copy(dst_h.at[wid], idxd_v)
    pltpu.sync_copy(g_h, gv_v)

    def _zrow(i, carry):
        for b in range(NF // 16):
            rows_s[0, i, pl.ds(b * 16, 16)] = jnp.zeros((16,), jnp.float32)
        exw_v[0, i, pl.ds(0, DW)] = jnp.zeros((DW,), jnp.float32)
        return carry
    lax.fori_loop(0, CH, _zrow, 0)
    for kk in range(4):
        pltpu.async_copy(rows_s.at[0], num_sh.at[pl.ds(r0 + kk * CH, CH)], sem_i)
        pltpu.async_copy(exw_v.at[0], den_sh.at[pl.ds(r0 + kk * CH, CH)], sem_i)
    pltpu.async_copy(rows_s.at[0].at[pl.ds(0, REM)],
                     num_sh.at[pl.ds(r0 + 4 * CH, REM)], sem_i)
    pltpu.async_copy(exw_v.at[0].at[pl.ds(0, REM)],
                     den_sh.at[pl.ds(r0 + 4 * CH, REM)], sem_i)
    for kk in range(4):
        pltpu.make_async_copy(rows_s.at[0], num_sh.at[pl.ds(r0 + kk * CH, CH)], sem_i).wait()
        pltpu.make_async_copy(exw_v.at[0], den_sh.at[pl.ds(r0 + kk * CH, CH)], sem_i).wait()
    pltpu.make_async_copy(rows_s.at[0].at[pl.ds(0, REM)],
                          num_sh.at[pl.ds(r0 + 4 * CH, REM)], sem_i).wait()
    pltpu.make_async_copy(exw_v.at[0].at[pl.ds(0, REM)],
                          den_sh.at[pl.ds(r0 + 4 * CH, REM)], sem_i).wait()

    plsc.subcore_barrier()

    gvec = gv_v[...]

    def _issue_gather(j, b, sem):
        pltpu.async_copy(as_h.at[idxs_v.at[j]], asg_a.at[b], sem)
        pltpu.async_copy(q_h.at[idxd_v.at[j]], qg_a.at[b], sem)
        pltpu.async_copy(h_h.at[idxs_v.at[j]], rows_g.at[b], sem)

    def _wait_gather_scalars(j, b, sem):
        pltpu.make_async_copy(as_h.at[idxs_v.at[j]], asg_a.at[b], sem).wait()
        pltpu.make_async_copy(q_h.at[idxd_v.at[j]], qg_a.at[b], sem).wait()

    def _wait_gather_rows(j, b, sem):
        pltpu.make_async_copy(h_h.at[idxs_v.at[j]], rows_g.at[b], sem).wait()

    def _wait_scatter(j, b, sem):
        pltpu.make_async_copy(rows_s.at[b], num_sh.at[idxd_v.at[j]], sem).wait()
        pltpu.make_async_copy(exw_v.at[b], den_sh.at[idxd_v.at[j]], sem).wait()

    _issue_gather(0, 0, sem_g0)
    _issue_gather(1, 1, sem_g1)

    def _pair(p, carry):
        for b in range(2):
            j = p * 2 + b
            sem_gb = sem_g0 if b == 0 else sem_g1
            sem_sb = sem_s0 if b == 0 else sem_s1
            _wait_gather_scalars(j, b, sem_gb)

            @pl.when(p > 0)
            def _():
                _wait_scatter(j, b, sem_sb)

            def _exk(k, c2):
                a = asg_a[b, pl.ds(k * 16, 16)]
                q = qg_a[b, pl.ds(k * 16, 16)]
                e = a + q
                e = jnp.where(e > 0.0, e, 0.2 * e)
                u = q + gvec
                mp = jnp.where(u > 0.0, u, 0.2 * u)
                ex_a[b, pl.ds(k * 16, 16)] = jnp.exp(e - mp)
                return c2
            lax.fori_loop(0, CH // 16, _exk, 0)

            def _exw(g, c2):
                exv = ex_a[b, pl.ds(g * 16, 16)]
                for i in range(16):
                    exw_v[b, g * 16 + i, pl.ds(0, DW)] = jnp.broadcast_to(exv[i], (DW,))
                return c2
            lax.fori_loop(0, CH // 16, _exw, 0)
            pltpu.async_copy(exw_v.at[b], den_sh.at[idxd_v.at[j]], sem_sb, add=True)

            _wait_gather_rows(j, b, sem_gb)

            def _scale(g, c2):
                exv = ex_a[b, pl.ds(g * 16, 16)]
                for i in range(16):
                    r = g * 16 + i
                    for bb in range(NF // 16):
                        rows_s[b, r, pl.ds(bb * 16, 16)] = (
                            rows_g[b, r, pl.ds(bb * 16, 16)] * exv[i])
                return c2
            lax.fori_loop(0, CH // 16, _scale, 0)

            @pl.when(j + 2 < NCH)
            def _():
                _issue_gather(j + 2, b, sem_gb)

            pltpu.async_copy(rows_s.at[b], num_sh.at[idxd_v.at[j]], sem_sb, add=True)
        return carry
    lax.fori_loop(0, NCH // 2, _pair, 0)

    _wait_scatter(NCH - 2, 0, sem_s0)
    _wait_scatter(NCH - 1, 1, sem_s1)

    plsc.subcore_barrier()

    o0 = c * RPAD + r0
    for kk in range(4):
        b = kk % 2
        pltpu.sync_copy(num_sh.at[pl.ds(r0 + kk * CH, CH)], rows_s.at[b])
        pltpu.async_copy(rows_s.at[b], num_o.at[pl.ds(o0 + kk * CH, CH)], sem_i)
        pltpu.sync_copy(den_sh.at[pl.ds(r0 + kk * CH, CH)], exw_v.at[b])
        pltpu.async_copy(exw_v.at[b], den_o.at[pl.ds(o0 + kk * CH, CH)], sem_i)
        if kk >= 1:
            pltpu.make_async_copy(rows_s.at[1 - b], num_o.at[pl.ds(o0 + (kk - 1) * CH, CH)], sem_i).wait()
            pltpu.make_async_copy(exw_v.at[1 - b], den_o.at[pl.ds(o0 + (kk - 1) * CH, CH)], sem_i).wait()
    pltpu.make_async_copy(rows_s.at[1], num_o.at[pl.ds(o0 + 3 * CH, CH)], sem_i).wait()
    pltpu.make_async_copy(exw_v.at[1], den_o.at[pl.ds(o0 + 3 * CH, CH)], sem_i).wait()
    pltpu.sync_copy(num_sh.at[pl.ds(r0 + 4 * CH, REM)], rows_s.at[0].at[pl.ds(0, REM)])
    pltpu.async_copy(rows_s.at[0].at[pl.ds(0, REM)], num_o.at[pl.ds(o0 + 4 * CH, REM)], sem_i)
    pltpu.sync_copy(den_sh.at[pl.ds(r0 + 4 * CH, REM)], exw_v.at[0].at[pl.ds(0, REM)])
    pltpu.async_copy(exw_v.at[0].at[pl.ds(0, REM)], den_o.at[pl.ds(o0 + 4 * CH, REM)], sem_i)
    pltpu.make_async_copy(rows_s.at[0].at[pl.ds(0, REM)], num_o.at[pl.ds(o0 + 4 * CH, REM)], sem_i).wait()
    pltpu.make_async_copy(exw_v.at[0].at[pl.ds(0, REM)], den_o.at[pl.ds(o0 + 4 * CH, REM)], sem_i).wait()


_sc_call = functools.partial(
    pl.kernel,
    out_type=(jax.ShapeDtypeStruct((NCORE * RPAD, NF), jnp.float32),
              jax.ShapeDtypeStruct((NCORE * RPAD, DW), jnp.float32)),
    mesh=plsc.VectorSubcoreMesh(core_axis_name="c", subcore_axis_name="s"),
    compiler_params=pltpu.CompilerParams(use_tc_tiling_on_sc=False),
    scratch_types=[
        pltpu.VMEM((NCH, CH), jnp.int32),
        pltpu.VMEM((NCH, CH), jnp.int32),
        pltpu.VMEM((2, CH), jnp.float32),
        pltpu.VMEM((2, CH), jnp.float32),
        pltpu.VMEM((2, CH), jnp.float32),
        pltpu.VMEM((2, CH, DW), jnp.float32),
        pltpu.VMEM((2, CH, NF), jnp.float32),
        pltpu.VMEM((2, CH, NF), jnp.float32),
        pltpu.VMEM((16,), jnp.float32),
        pltpu.VMEM_SHARED((RPAD, NF), jnp.float32),
        pltpu.VMEM_SHARED((RPAD, DW), jnp.float32),
        pltpu.SemaphoreType.DMA,
        pltpu.SemaphoreType.DMA,
        pltpu.SemaphoreType.DMA,
        pltpu.SemaphoreType.DMA,
        pltpu.SemaphoreType.DMA,
    ],
)(_sc_gat)


BLKA = 2000
BLKB = 1264
BLKG = 2000
BLKO = 2000


def _tc_pre(px_ref, wgat_ref, a2_ref, w_ref, gsf_ref,
            h_ref, asq_ref, g_ref, w_out_ref):
    i = pl.program_id(0)
    x = px_ref[...]
    x = x - jnp.max(x, axis=1, keepdims=True)
    e = jnp.exp(x)
    f = e / jnp.sum(e, axis=1, keepdims=True)
    h = jnp.dot(f, wgat_ref[...], preferred_element_type=jnp.float32)
    h_ref[...] = h
    asq = jnp.dot(h, a2_ref[...], preferred_element_type=jnp.float32)
    asq_ref[...] = asq
    cur = jnp.max(asq[:, 0])
    prev = g_ref[0, 0]
    g_ref[0, 0] = jnp.where(i == 0, cur, jnp.maximum(prev, cur))

    @pl.when(i == 0)
    def _():
        wv = w_ref[...]
        wv = wv - jnp.max(wv, axis=1, keepdims=True)
        ew = jnp.exp(wv)
        ws = ew / jnp.sum(ew, axis=1, keepdims=True)
        w_out_ref[...] = ws * jnp.exp(gsf_ref[...])


def _tc_norm(num_ref, den_ref, bg_ref, xf_ref):
    n = num_ref[0] + num_ref[1]
    d = den_ref[0] + den_ref[1] + 1e-16
    x = n / d + bg_ref[...]
    x = x - jnp.max(x, axis=1, keepdims=True)
    e = jnp.exp(x)
    xf_ref[...] = e / jnp.sum(e, axis=1, keepdims=True)


def _tc_gamma(gamma_ref, xf_ref, vf_ref, acc_ref):
    i = pl.program_id(0)
    part = lax.dot_general(gamma_ref[...], xf_ref[...],
                           (((0,), (0,)), ((), ())),
                           preferred_element_type=jnp.float32)

    @pl.when(i == 0)
    def _():
        acc_ref[...] = part

    @pl.when(i > 0)
    def _():
        acc_ref[...] = acc_ref[...] + part

    @pl.when(i == pl.num_programs(0) - 1)
    def _():
        a = acc_ref[...]
        a = a - jnp.max(a, axis=1, keepdims=True)
        e = jnp.exp(a)
        vf_ref[...] = e / jnp.sum(e, axis=1, keepdims=True)


def _tc_final(xf_ref, vf_ref, w_ref, sfv_ref, out_ref):
    i = pl.program_id(0)

    @pl.when(i < NXE // BLKO)
    def _():
        out_ref[...] = jnp.dot(xf_ref[...], w_ref[...],
                               preferred_element_type=jnp.float32)

    @pl.when(i == NXE // BLKO)
    def _():
        out_ref[...] = jnp.exp(sfv_ref[...]) * jnp.dot(
            vf_ref[...], w_ref[...], preferred_element_type=jnp.float32)


def kernel(Px, W, gene_sf, scale_factor_visium, gamma,
           W_gat, a_src, a_dst, b_gat, edge_index):
    a2 = jnp.stack([a_src, a_dst], axis=1)

    h, asq, gmax, w = pl.pallas_call(
        _tc_pre,
        grid=(NXE // BLKA,),
        in_specs=[
            pl.BlockSpec((BLKA, NF), lambda i: (i, 0)),
            pl.BlockSpec((NF, NF), lambda i: (0, 0)),
            pl.BlockSpec((NF, 2), lambda i: (0, 0)),
            pl.BlockSpec((NF, NG), lambda i: (0, 0)),
            pl.BlockSpec((1, NG), lambda i: (0, 0)),
        ],
        out_specs=[
            pl.BlockSpec((BLKA, NF), lambda i: (i, 0)),
            pl.BlockSpec((BLKA, 2), lambda i: (i, 0)),
            pl.BlockSpec(memory_space=pltpu.SMEM),
            pl.BlockSpec((NF, NG), lambda i: (0, 0)),
        ],
        out_shape=[
            jax.ShapeDtypeStruct((NXE, NF), jnp.float32),
            jax.ShapeDtypeStruct((NXE, 2), jnp.float32),
            jax.ShapeDtypeStruct((1, 1), jnp.float32),
            jax.ShapeDtypeStruct((NF, NG), jnp.float32),
        ],
    )(Px, W_gat, a2, W, gene_sf)

    as_p = asq[:, 0]
    q_p = jnp.pad(asq[:, 1], (0, 8))
    g16 = jnp.broadcast_to(gmax.reshape(1), (16,))

    src_p = jnp.concatenate(
        [edge_index[0], jnp.zeros((EPAD - NE,), jnp.int32)]).reshape(NW, NCH, CH)
    dst_p = jnp.concatenate(
        [edge_index[1], jnp.full((EPAD - NE,), NXE, jnp.int32)]).reshape(NW, NCH, CH)

    num2, den2 = _sc_call(src_p, dst_p, as_p, q_p, h, g16)
    num2 = num2.reshape(NCORE, RPAD, NF)
    den2 = den2.reshape(NCORE, RPAD, DW)[:, :, :1]

    xf_full = pl.pallas_call(
        _tc_norm,
        grid=(RPAD // BLKB,),
        in_specs=[
            pl.BlockSpec((NCORE, BLKB, NF), lambda i: (0, i, 0)),
            pl.BlockSpec((NCORE, BLKB, 1), lambda i: (0, i, 0)),
            pl.BlockSpec((1, NF), lambda i: (0, 0)),
        ],
        out_specs=pl.BlockSpec((BLKB, NF), lambda i: (i, 0)),
        out_shape=jax.ShapeDtypeStruct((RPAD, NF), jnp.float32),
    )(num2, den2, b_gat.reshape(1, NF))
    xf = xf_full

    vf = pl.pallas_call(
        _tc_gamma,
        grid=(NXE // BLKG,),
        in_specs=[
            pl.BlockSpec((BLKG, NVI), lambda i: (i, 0)),
            pl.BlockSpec((BLKG, NF), lambda i: (i, 0)),
        ],
        out_specs=pl.BlockSpec((NVI, NF), lambda i: (0, 0)),
        out_shape=jax.ShapeDtypeStruct((NVI, NF), jnp.float32),
        scratch_shapes=[pltpu.VMEM((NVI, NF), jnp.float32)],
    )(gamma, xf)

    out = pl.pallas_call(
        _tc_final,
        grid=(NXE // BLKO + 1,),
        in_specs=[
            pl.BlockSpec((BLKO, NF), lambda i: (jnp.minimum(i, NXE // BLKO - 1), 0)),
            pl.BlockSpec((NVI, NF), lambda i: (0, 0)),
            pl.BlockSpec((NF, NG), lambda i: (0, 0)),
            pl.BlockSpec((NVI, 1), lambda i: (0, 0)),
        ],
        out_specs=pl.BlockSpec((BLKO, NG), lambda i: (i, 0)),
        out_shape=jax.ShapeDtypeStruct((NXE + NVI, NG), jnp.float32),
    )(xf, vf, w, scale_factor_visium)

    return out
